# baseline (speedup 1.0000x reference)
import jax
import jax.numpy as jnp
from jax.experimental import pallas as pl
from jax.experimental.pallas import tpu as pltpu

B = 32
L = 2048
D_IN = 128
EMBED_DIM = 128
CHUNK = 2048
NCHUNK = (B * L) // CHUNK
PHASE = 16
NPHASE = NCHUNK // PHASE


def _fused_kernel(x_hbm, w_ref, b_ref, pos_ref, out_hbm,
                  in_buf, out_buf, in_sems, out_sems):
    def in_copy(c, j):
        return pltpu.make_async_copy(
            x_hbm.at[pl.ds(c * CHUNK, CHUNK), :], in_buf.at[j], in_sems.at[j])

    def out_copy(c, j):
        return pltpu.make_async_copy(
            out_buf.at[j], out_hbm.at[pl.ds(c * CHUNK, CHUNK), :], out_sems.at[j])

    pos_bias = pos_ref[...] + b_ref[...]
    for s in range(NPHASE):
        if s > 0:
            for j in range(PHASE):
                out_copy((s - 1) * PHASE + j, j).wait()
        for j in range(PHASE):
            in_copy(s * PHASE + j, j).start()
        for j in range(PHASE):
            in_copy(s * PHASE + j, j).wait()
            acc = jnp.dot(in_buf[j], w_ref[...],
                          preferred_element_type=jnp.float32)
            out_buf[j] = acc + pos_bias
        for j in range(PHASE):
            out_copy(s * PHASE + j, j).start()
    for j in range(PHASE):
        out_copy((NPHASE - 1) * PHASE + j, j).wait()


def kernel(x, W, b, pos_table):
    x2 = x.reshape(B * L, D_IN)
    b2 = b.reshape(1, EMBED_DIM)
    out = pl.pallas_call(
        _fused_kernel,
        in_specs=[
            pl.BlockSpec(memory_space=pltpu.MemorySpace.HBM),
            pl.BlockSpec(memory_space=pltpu.MemorySpace.VMEM),
            pl.BlockSpec(memory_space=pltpu.MemorySpace.VMEM),
            pl.BlockSpec(memory_space=pltpu.MemorySpace.VMEM),
        ],
        out_specs=pl.BlockSpec(memory_space=pltpu.MemorySpace.HBM),
        out_shape=jax.ShapeDtypeStruct((B * L, EMBED_DIM), jnp.float32),
        scratch_shapes=[
            pltpu.MemorySpace.VMEM((PHASE, CHUNK, D_IN), jnp.float32),
            pltpu.MemorySpace.VMEM((PHASE, CHUNK, EMBED_DIM), jnp.float32),
            pltpu.SemaphoreType.DMA((PHASE,)),
            pltpu.SemaphoreType.DMA((PHASE,)),
        ],
    )(x2, W, b2, pos_table)
    return out.reshape(B, L, EMBED_DIM)


# speedup vs baseline: 1.3193x; 1.3193x over previous
import jax
import jax.numpy as jnp
from jax.experimental import pallas as pl

B = 32
L = 2048
D_IN = 128
EMBED_DIM = 128
BLK = 16384


def _fused_kernel(x_ref, w_ref, b_ref, pos_ref, out_ref):
    acc = jnp.dot(x_ref[...], w_ref[...], preferred_element_type=jnp.float32)
    m = BLK // L
    acc = acc.reshape(m, L, EMBED_DIM) + pos_ref[...][None, :, :] + b_ref[...]
    out_ref[...] = acc.reshape(BLK, EMBED_DIM)


def kernel(x, W, b, pos_table):
    x2 = x.reshape(B * L, D_IN)
    b2 = b.reshape(1, EMBED_DIM)
    assert (B * L) % BLK == 0 and BLK % L == 0
    grid = (B * L) // BLK
    out = pl.pallas_call(
        _fused_kernel,
        grid=(grid,),
        in_specs=[
            pl.BlockSpec((BLK, D_IN), lambda i: (i, 0)),
            pl.BlockSpec((D_IN, EMBED_DIM), lambda i: (0, 0)),
            pl.BlockSpec((1, EMBED_DIM), lambda i: (0, 0)),
            pl.BlockSpec((L, EMBED_DIM), lambda i: (0, 0)),
        ],
        out_specs=pl.BlockSpec((BLK, EMBED_DIM), lambda i: (i, 0)),
        out_shape=jax.ShapeDtypeStruct((B * L, EMBED_DIM), jnp.float32),
    )(x2, W, b2, pos_table)
    return out.reshape(B, L, EMBED_DIM)
